# baseline (speedup 1.0000x reference)
import jax
import jax.numpy as jnp
from jax.experimental import pallas as pl
from jax.experimental.pallas import tpu as pltpu


def _ring_slot_copy(src_ref, dst_ref, sem):
    copy = pltpu.make_async_copy(src_ref, dst_ref, sem)
    copy.start()
    copy.wait()


def kernel(img_batch, canvases):
    del canvases
    return pl.pallas_call(
        _ring_slot_copy,
        out_shape=jax.ShapeDtypeStruct(img_batch.shape, img_batch.dtype),
        in_specs=[pl.BlockSpec(memory_space=pltpu.MemorySpace.HBM)],
        out_specs=pl.BlockSpec(memory_space=pltpu.MemorySpace.HBM),
        scratch_shapes=[pltpu.SemaphoreType.DMA],
    )(img_batch)


# speedup vs baseline: 1.0001x; 1.0001x over previous
import jax
import jax.numpy as jnp
from jax.experimental import pallas as pl
from jax.experimental.pallas import tpu as pltpu

_NCHUNKS = 16


def _ring_slot_copy(src_ref, dst_ref, sems):
    rows = src_ref.shape[0]
    chunk = rows // _NCHUNKS
    for i in range(_NCHUNKS):
        pltpu.make_async_copy(
            src_ref.at[pl.ds(i * chunk, chunk)],
            dst_ref.at[pl.ds(i * chunk, chunk)],
            sems.at[i],
        ).start()
    for i in range(_NCHUNKS):
        pltpu.make_async_copy(
            src_ref.at[pl.ds(i * chunk, chunk)],
            dst_ref.at[pl.ds(i * chunk, chunk)],
            sems.at[i],
        ).wait()


def kernel(img_batch, canvases):
    del canvases
    b, c, h, w = img_batch.shape
    flat = img_batch.reshape(b * c * h, w)
    out = pl.pallas_call(
        _ring_slot_copy,
        out_shape=jax.ShapeDtypeStruct(flat.shape, flat.dtype),
        in_specs=[pl.BlockSpec(memory_space=pltpu.MemorySpace.HBM)],
        out_specs=pl.BlockSpec(memory_space=pltpu.MemorySpace.HBM),
        scratch_shapes=[pltpu.SemaphoreType.DMA((_NCHUNKS,))],
    )(flat)
    return out.reshape(b, c, h, w)


# speedup vs baseline: 42.6620x; 42.6596x over previous
import jax
import jax.numpy as jnp
from jax.experimental import pallas as pl
from jax.experimental.pallas import tpu as pltpu

_BLOCK_ROWS = 1024


def _ring_slot_copy(src_ref, dst_ref):
    dst_ref[...] = src_ref[...]


def kernel(img_batch, canvases):
    del canvases
    b, c, h, w = img_batch.shape
    flat = img_batch.reshape(b * c * h, w)
    rows = flat.shape[0]
    grid = rows // _BLOCK_ROWS
    out = pl.pallas_call(
        _ring_slot_copy,
        grid=(grid,),
        in_specs=[pl.BlockSpec((_BLOCK_ROWS, w), lambda i: (i, 0))],
        out_specs=pl.BlockSpec((_BLOCK_ROWS, w), lambda i: (i, 0)),
        out_shape=jax.ShapeDtypeStruct(flat.shape, flat.dtype),
    )(flat)
    return out.reshape(b, c, h, w)


# speedup vs baseline: 46.6224x; 1.0928x over previous
import jax
import jax.numpy as jnp
from jax.experimental import pallas as pl
from jax.experimental.pallas import tpu as pltpu

_BLOCK_ROWS = 2048


def _ring_slot_copy(src_ref, dst_ref):
    dst_ref[...] = src_ref[...]


def kernel(img_batch, canvases):
    del canvases
    b, c, h, w = img_batch.shape
    flat = img_batch.reshape(b * c * h, w)
    rows = flat.shape[0]
    grid = rows // _BLOCK_ROWS
    out = pl.pallas_call(
        _ring_slot_copy,
        grid=(grid,),
        in_specs=[pl.BlockSpec((_BLOCK_ROWS, w), lambda i: (i, 0))],
        out_specs=pl.BlockSpec((_BLOCK_ROWS, w), lambda i: (i, 0)),
        out_shape=jax.ShapeDtypeStruct(flat.shape, flat.dtype),
    )(flat)
    return out.reshape(b, c, h, w)


# speedup vs baseline: 49.1252x; 1.0537x over previous
import jax
import jax.numpy as jnp
from jax.experimental import pallas as pl
from jax.experimental.pallas import tpu as pltpu

_BLOCK_ROWS = 4096


def _ring_slot_copy(src_ref, dst_ref):
    dst_ref[...] = src_ref[...]


def kernel(img_batch, canvases):
    del canvases
    b, c, h, w = img_batch.shape
    flat = img_batch.reshape(b * c * h, w)
    rows = flat.shape[0]
    grid = rows // _BLOCK_ROWS
    out = pl.pallas_call(
        _ring_slot_copy,
        grid=(grid,),
        in_specs=[pl.BlockSpec((_BLOCK_ROWS, w), lambda i: (i, 0))],
        out_specs=pl.BlockSpec((_BLOCK_ROWS, w), lambda i: (i, 0)),
        out_shape=jax.ShapeDtypeStruct(flat.shape, flat.dtype),
    )(flat)
    return out.reshape(b, c, h, w)


# speedup vs baseline: 49.5374x; 1.0084x over previous
import jax
import jax.numpy as jnp
from jax.experimental import pallas as pl
from jax.experimental.pallas import tpu as pltpu

_BLOCK_ROWS = 6144


def _ring_slot_copy(src_ref, dst_ref):
    dst_ref[...] = src_ref[...]


def kernel(img_batch, canvases):
    del canvases
    b, c, h, w = img_batch.shape
    flat = img_batch.reshape(b * c * h, w)
    rows = flat.shape[0]
    grid = rows // _BLOCK_ROWS
    out = pl.pallas_call(
        _ring_slot_copy,
        grid=(grid,),
        in_specs=[pl.BlockSpec((_BLOCK_ROWS, w), lambda i: (i, 0))],
        out_specs=pl.BlockSpec((_BLOCK_ROWS, w), lambda i: (i, 0)),
        out_shape=jax.ShapeDtypeStruct(flat.shape, flat.dtype),
    )(flat)
    return out.reshape(b, c, h, w)
